# baseline (speedup 1.0000x reference)
import functools

import jax
import jax.numpy as jnp
from jax import lax
from jax.experimental import pallas as pl
from jax.experimental.pallas import tpu as pltpu
from jax.experimental.pallas import tpu_sc as plsc

N = 10000
C = 128
OUT = 64
N_PAD = 10240
NTILES = 16
NCORES = 2
NW = NCORES * NTILES
CHUNK = 128
CH = 79
E_PAD = NW * CH * CHUNK
ROWS_PER_TILE = N_PAD // NTILES

_MESH = plsc.VectorSubcoreMesh(core_axis_name="c", subcore_axis_name="s")


def _zero_vmem(ref, nrows, width):
    def zrow(i, _):
        def zcol(k, _2):
            ref[i, pl.ds(k * 16, 16)] = jnp.zeros((16,), jnp.float32)
            return 0
        return lax.fori_loop(0, width // 16, zcol, 0)
    lax.fori_loop(0, nrows, zrow, 0)


def _sc_deg_body(dst_hbm, out_hbm, dst_v, ones_v, zbuf, acc):
    cid = lax.axis_index("c")
    sid = lax.axis_index("s")
    wid = cid * NTILES + sid
    _zero_vmem(zbuf, 128, 16)
    def orow(i, _):
        ones_v[i, :] = jnp.ones((16,), jnp.float32)
        return 0
    lax.fori_loop(0, CHUNK, orow, 0)
    def zacc(k, _):
        pltpu.sync_copy(zbuf, acc.at[pl.ds(sid * ROWS_PER_TILE + k * 128, 128)])
        return 0
    lax.fori_loop(0, ROWS_PER_TILE // 128, zacc, 0)
    pltpu.sync_copy(dst_hbm.at[wid], dst_v)
    plsc.subcore_barrier()
    def chunk(j, _):
        pltpu.sync_copy(ones_v, acc.at[dst_v.at[j]], add=True)
        return 0
    lax.fori_loop(0, CH, chunk, 0)
    plsc.subcore_barrier()
    pltpu.sync_copy(acc.at[pl.ds(sid * ROWS_PER_TILE, ROWS_PER_TILE)],
                    out_hbm.at[cid, pl.ds(sid * ROWS_PER_TILE, ROWS_PER_TILE)])


_sc_deg = pl.kernel(
    _sc_deg_body,
    out_type=jax.ShapeDtypeStruct((NCORES, N_PAD, 16), jnp.float32),
    mesh=_MESH,
    scratch_types=[
        pltpu.VMEM((CH, CHUNK), jnp.int32),
        pltpu.VMEM((CHUNK, 16), jnp.float32),
        pltpu.VMEM((128, 16), jnp.float32),
        pltpu.VMEM_SHARED((N_PAD, 16), jnp.float32),
    ],
)


def _sc_agg_body(xs_hbm, ei_hbm, out_hbm,
                 idxA, idxB, rowsA, rowsB, zbuf,
                 gsemA, gsemB, isemA, isemB, acc):
    cid = lax.axis_index("c")
    sid = lax.axis_index("s")
    wid = cid * NTILES + sid
    _zero_vmem(zbuf, 16, C)
    def zacc(k, _):
        pltpu.sync_copy(zbuf, acc.at[pl.ds(sid * ROWS_PER_TILE + k * 16, 16)])
        return 0
    lax.fori_loop(0, ROWS_PER_TILE // 16, zacc, 0)
    plsc.subcore_barrier()
    pltpu.async_copy(ei_hbm.at[wid, 0], idxA, isemA).wait()
    pltpu.async_copy(xs_hbm.at[idxA.at[0]], rowsA, gsemA)
    pltpu.async_copy(ei_hbm.at[wid, 1], idxB, isemB)

    def half(j, idx_f, rows_f, gsem_f, isem_f, idx_n, rows_n, gsem_n, isem_n):
        @pl.when(j + 1 < CH)
        def _():
            pltpu.make_async_copy(ei_hbm.at[wid, 0], idx_n, isem_n).wait()
            pltpu.async_copy(xs_hbm.at[idx_n.at[0]], rows_n, gsem_n)
        pltpu.make_async_copy(xs_hbm.at[idx_f.at[0]], rows_f, gsem_f).wait()
        pltpu.sync_copy(rows_f, acc.at[idx_f.at[1]], add=True)
        @pl.when(j + 2 < CH)
        def _():
            pltpu.async_copy(ei_hbm.at[wid, j + 2], idx_f, isem_f)

    def pair(i, _):
        j0 = 2 * i
        half(j0, idxA, rowsA, gsemA, isemA, idxB, rowsB, gsemB, isemB)
        half(j0 + 1, idxB, rowsB, gsemB, isemB, idxA, rowsA, gsemA, isemA)
        return 0
    lax.fori_loop(0, CH // 2, pair, 0)
    pltpu.make_async_copy(xs_hbm.at[idxA.at[0]], rowsA, gsemA).wait()
    pltpu.sync_copy(rowsA, acc.at[idxA.at[1]], add=True)
    plsc.subcore_barrier()
    pltpu.sync_copy(acc.at[pl.ds(sid * ROWS_PER_TILE, ROWS_PER_TILE)],
                    out_hbm.at[cid, pl.ds(sid * ROWS_PER_TILE, ROWS_PER_TILE)])


_sc_agg = pl.kernel(
    _sc_agg_body,
    out_type=jax.ShapeDtypeStruct((NCORES, N_PAD, C), jnp.float32),
    mesh=_MESH,
    scratch_types=[
        pltpu.VMEM((2, CHUNK), jnp.int32),
        pltpu.VMEM((2, CHUNK), jnp.int32),
        pltpu.VMEM((CHUNK, C), jnp.float32),
        pltpu.VMEM((CHUNK, C), jnp.float32),
        pltpu.VMEM((16, C), jnp.float32),
        pltpu.SemaphoreType.DMA,
        pltpu.SemaphoreType.DMA,
        pltpu.SemaphoreType.DMA,
        pltpu.SemaphoreType.DMA,
        pltpu.VMEM_SHARED((N_PAD, C), jnp.float32),
    ],
)


R = 1000
G = N // R


def _tc_prescale_body(d0, d1, x, xs_o, dis_o):
    deg = d0[...] + d1[...] + 1.0
    dis = lax.rsqrt(deg)
    dis_o[...] = dis
    xs_o[...] = x[...] * dis


_tc_prescale = pl.pallas_call(
    _tc_prescale_body,
    grid=(G,),
    in_specs=[
        pl.BlockSpec((R, 1), lambda i: (i, 0)),
        pl.BlockSpec((R, 1), lambda i: (i, 0)),
        pl.BlockSpec((R, C), lambda i: (i, 0)),
    ],
    out_specs=[
        pl.BlockSpec((R, C), lambda i: (i, 0)),
        pl.BlockSpec((R, 1), lambda i: (i, 0)),
    ],
    out_shape=[
        jax.ShapeDtypeStruct((N, C), jnp.float32),
        jax.ShapeDtypeStruct((N, 1), jnp.float32),
    ],
)


_Y_SPECS = [
    pl.BlockSpec((NCORES, R, C), lambda i: (0, i, 0)),
]


def _tc_mid_body(p, base, dis, W, b, o_ref):
    disv = dis[...]
    pv = p[...]
    z = (pv[0] + pv[1] + base[...]) * disv
    o = jnp.dot(z, W[...], preferred_element_type=jnp.float32,
                precision=lax.Precision.HIGHEST) + b[...][None, :]
    o_ref[...] = jnp.maximum(o, 0.0) * disv


_tc_mid = pl.pallas_call(
    _tc_mid_body,
    grid=(G,),
    in_specs=_Y_SPECS + [
        pl.BlockSpec((R, C), lambda i: (i, 0)),
        pl.BlockSpec((R, 1), lambda i: (i, 0)),
        pl.BlockSpec((C, C), lambda i: (0, 0)),
        pl.BlockSpec((C,), lambda i: (0,)),
    ],
    out_specs=pl.BlockSpec((R, C), lambda i: (i, 0)),
    out_shape=jax.ShapeDtypeStruct((N, C), jnp.float32),
)


def _tc_final_body(p, base, dis, W, b, mu_ref, ls_ref):
    disv = dis[...]
    pv = p[...]
    z = (pv[0] + pv[1] + base[...]) * disv
    o = jnp.dot(z, W[...], preferred_element_type=jnp.float32,
                precision=lax.Precision.HIGHEST) + b[...][None, :]
    mu_ref[...] = o[:, :OUT]
    ls_ref[...] = o[:, OUT:]


_tc_final = pl.pallas_call(
    _tc_final_body,
    grid=(G,),
    in_specs=_Y_SPECS + [
        pl.BlockSpec((R, C), lambda i: (i, 0)),
        pl.BlockSpec((R, 1), lambda i: (i, 0)),
        pl.BlockSpec((C, C), lambda i: (0, 0)),
        pl.BlockSpec((C,), lambda i: (0,)),
    ],
    out_specs=[
        pl.BlockSpec((R, OUT), lambda i: (i, 0)),
        pl.BlockSpec((R, OUT), lambda i: (i, 0)),
    ],
    out_shape=[
        jax.ShapeDtypeStruct((N, OUT), jnp.float32),
        jax.ShapeDtypeStruct((N, OUT), jnp.float32),
    ],
)


def kernel(x, edge_index, W1, b1, W_mu, b_mu, W_ls, b_ls):
    ei = edge_index.astype(jnp.int32)
    src, dst = ei[0], ei[1]
    pad = E_PAD - src.shape[0]
    src_f = jnp.concatenate([src, jnp.zeros((pad,), jnp.int32)])
    dst_f = jnp.concatenate([dst, jnp.full((pad,), N, jnp.int32)])
    dst_p = dst_f.reshape(NW, CH, CHUNK)
    nchunks = E_PAD // CHUNK
    ei_p = jnp.stack([src_f.reshape(nchunks, CHUNK),
                      dst_f.reshape(nchunks, CHUNK)],
                     axis=1).reshape(NW, CH, 2, CHUNK)

    degp = _sc_deg(dst_p)
    d0 = degp[0, :N, 0:1]
    d1 = degp[1, :N, 0:1]
    xs, dis = _tc_prescale(d0, d1, x)

    y1 = _sc_agg(xs, ei_p)
    hs = _tc_mid(y1, xs, dis, W1, b1)

    y2 = _sc_agg(hs, ei_p)
    Wc = jnp.concatenate([W_mu, W_ls], axis=1)
    bc = jnp.concatenate([b_mu, b_ls])
    mu, ls = _tc_final(y2, hs, dis, Wc, bc)
    return mu, ls


# speedup vs baseline: 1.0108x; 1.0108x over previous
import functools

import jax
import jax.numpy as jnp
from jax import lax
from jax.experimental import pallas as pl
from jax.experimental.pallas import tpu as pltpu
from jax.experimental.pallas import tpu_sc as plsc

N = 10000
C = 128
OUT = 64
N_PAD = 10112
NTILES = 16
NCORES = 2
NW = NCORES * NTILES
CHUNK = 128
CH = 79
E_PAD = NW * CH * CHUNK
ROWS_PER_TILE = N_PAD // NTILES

_MESH = plsc.VectorSubcoreMesh(core_axis_name="c", subcore_axis_name="s")


def _zero_vmem(ref, nrows, width):
    def zrow(i, _):
        def zcol(k, _2):
            ref[i, pl.ds(k * 16, 16)] = jnp.zeros((16,), jnp.float32)
            return 0
        return lax.fori_loop(0, width // 16, zcol, 0)
    lax.fori_loop(0, nrows, zrow, 0)


def _sc_deg_body(dst_hbm, out_hbm, dst_v, ones_v, zbuf, acc):
    cid = lax.axis_index("c")
    sid = lax.axis_index("s")
    wid = cid * NTILES + sid
    _zero_vmem(zbuf, 128, 16)
    def orow(i, _):
        ones_v[i, :] = jnp.ones((16,), jnp.float32)
        return 0
    lax.fori_loop(0, CHUNK, orow, 0)
    def zacc(k, _):
        pltpu.sync_copy(zbuf, acc.at[pl.ds(sid * ROWS_PER_TILE + k * 128, 128)])
        return 0
    lax.fori_loop(0, ROWS_PER_TILE // 128, zacc, 0)
    pltpu.sync_copy(zbuf.at[pl.ds(0, ROWS_PER_TILE % 128)],
                    acc.at[pl.ds(sid * ROWS_PER_TILE
                                 + (ROWS_PER_TILE // 128) * 128,
                                 ROWS_PER_TILE % 128)])
    pltpu.sync_copy(dst_hbm.at[wid], dst_v)
    plsc.subcore_barrier()
    def chunk(j, _):
        pltpu.sync_copy(ones_v, acc.at[dst_v.at[j]], add=True)
        return 0
    lax.fori_loop(0, CH, chunk, 0)
    plsc.subcore_barrier()
    pltpu.sync_copy(acc.at[pl.ds(sid * ROWS_PER_TILE, ROWS_PER_TILE)],
                    out_hbm.at[cid, pl.ds(sid * ROWS_PER_TILE, ROWS_PER_TILE)])


_sc_deg = pl.kernel(
    _sc_deg_body,
    out_type=jax.ShapeDtypeStruct((NCORES, N_PAD, 16), jnp.float32),
    mesh=_MESH,
    scratch_types=[
        pltpu.VMEM((CH, CHUNK), jnp.int32),
        pltpu.VMEM((CHUNK, 16), jnp.float32),
        pltpu.VMEM((128, 16), jnp.float32),
        pltpu.VMEM_SHARED((N_PAD, 16), jnp.float32),
    ],
)


def _sc_agg_body(xs_hbm, ei_hbm, out_hbm,
                 idxA, idxB, idxC, rowsA, rowsB, rowsC,
                 gsemA, gsemB, gsemC, isemA, isemB, isemC, acc):
    cid = lax.axis_index("c")
    sid = lax.axis_index("s")
    wid = cid * NTILES + sid
    _zero_vmem(rowsA, 128, C)
    def zacc(k, _):
        pltpu.sync_copy(rowsA, acc.at[pl.ds(sid * ROWS_PER_TILE + k * 128, 128)])
        return 0
    lax.fori_loop(0, ROWS_PER_TILE // 128, zacc, 0)
    pltpu.sync_copy(rowsA.at[pl.ds(0, ROWS_PER_TILE % 128)],
                    acc.at[pl.ds(sid * ROWS_PER_TILE
                                 + (ROWS_PER_TILE // 128) * 128,
                                 ROWS_PER_TILE % 128)])
    plsc.subcore_barrier()
    pltpu.async_copy(ei_hbm.at[wid, 0], idxA, isemA).wait()
    pltpu.async_copy(xs_hbm.at[idxA.at[0]], rowsA, gsemA)
    pltpu.async_copy(ei_hbm.at[wid, 1], idxB, isemB).wait()
    pltpu.async_copy(xs_hbm.at[idxB.at[0]], rowsB, gsemB)
    pltpu.async_copy(ei_hbm.at[wid, 2], idxC, isemC)

    def third(j, idx_c, rows_c, gsem_c, isem_c,
              idx_n2, rows_n2, gsem_n2, isem_n2):
        @pl.when(j + 2 < CH)
        def _():
            pltpu.make_async_copy(ei_hbm.at[wid, 0], idx_n2, isem_n2).wait()
            pltpu.async_copy(xs_hbm.at[idx_n2.at[0]], rows_n2, gsem_n2)
        pltpu.make_async_copy(xs_hbm.at[idx_c.at[0]], rows_c, gsem_c).wait()
        pltpu.sync_copy(rows_c, acc.at[idx_c.at[1]], add=True)
        @pl.when(j + 3 < CH)
        def _():
            pltpu.async_copy(ei_hbm.at[wid, j + 3], idx_c, isem_c)

    def triple(i, _):
        j0 = 3 * i
        third(j0, idxA, rowsA, gsemA, isemA, idxC, rowsC, gsemC, isemC)
        third(j0 + 1, idxB, rowsB, gsemB, isemB, idxA, rowsA, gsemA, isemA)
        third(j0 + 2, idxC, rowsC, gsemC, isemC, idxB, rowsB, gsemB, isemB)
        return 0
    lax.fori_loop(0, CH // 3, triple, 0)
    pltpu.make_async_copy(xs_hbm.at[idxA.at[0]], rowsA, gsemA).wait()
    pltpu.sync_copy(rowsA, acc.at[idxA.at[1]], add=True)
    plsc.subcore_barrier()
    pltpu.sync_copy(acc.at[pl.ds(sid * ROWS_PER_TILE, ROWS_PER_TILE)],
                    out_hbm.at[cid, pl.ds(sid * ROWS_PER_TILE, ROWS_PER_TILE)])


_sc_agg = pl.kernel(
    _sc_agg_body,
    out_type=jax.ShapeDtypeStruct((NCORES, N_PAD, C), jnp.float32),
    mesh=_MESH,
    scratch_types=[
        pltpu.VMEM((2, CHUNK), jnp.int32),
        pltpu.VMEM((2, CHUNK), jnp.int32),
        pltpu.VMEM((2, CHUNK), jnp.int32),
        pltpu.VMEM((CHUNK, C), jnp.float32),
        pltpu.VMEM((CHUNK, C), jnp.float32),
        pltpu.VMEM((CHUNK, C), jnp.float32),
        pltpu.SemaphoreType.DMA,
        pltpu.SemaphoreType.DMA,
        pltpu.SemaphoreType.DMA,
        pltpu.SemaphoreType.DMA,
        pltpu.SemaphoreType.DMA,
        pltpu.SemaphoreType.DMA,
        pltpu.VMEM_SHARED((N_PAD, C), jnp.float32),
    ],
)


R = 1000
G = N // R


def _tc_prescale_body(d0, d1, x, xs_o, dis_o):
    deg = d0[...] + d1[...] + 1.0
    dis = lax.rsqrt(deg)
    dis_o[...] = dis
    xs_o[...] = x[...] * dis


_tc_prescale = pl.pallas_call(
    _tc_prescale_body,
    grid=(G,),
    in_specs=[
        pl.BlockSpec((R, 1), lambda i: (i, 0)),
        pl.BlockSpec((R, 1), lambda i: (i, 0)),
        pl.BlockSpec((R, C), lambda i: (i, 0)),
    ],
    out_specs=[
        pl.BlockSpec((R, C), lambda i: (i, 0)),
        pl.BlockSpec((R, 1), lambda i: (i, 0)),
    ],
    out_shape=[
        jax.ShapeDtypeStruct((N, C), jnp.float32),
        jax.ShapeDtypeStruct((N, 1), jnp.float32),
    ],
)


_Y_SPECS = [
    pl.BlockSpec((NCORES, R, C), lambda i: (0, i, 0)),
]


def _tc_mid_body(p, base, dis, W, b, o_ref):
    disv = dis[...]
    pv = p[...]
    z = (pv[0] + pv[1] + base[...]) * disv
    o = jnp.dot(z, W[...], preferred_element_type=jnp.float32,
                precision=lax.Precision.HIGHEST) + b[...][None, :]
    o_ref[...] = jnp.maximum(o, 0.0) * disv


_tc_mid = pl.pallas_call(
    _tc_mid_body,
    grid=(G,),
    in_specs=_Y_SPECS + [
        pl.BlockSpec((R, C), lambda i: (i, 0)),
        pl.BlockSpec((R, 1), lambda i: (i, 0)),
        pl.BlockSpec((C, C), lambda i: (0, 0)),
        pl.BlockSpec((C,), lambda i: (0,)),
    ],
    out_specs=pl.BlockSpec((R, C), lambda i: (i, 0)),
    out_shape=jax.ShapeDtypeStruct((N, C), jnp.float32),
)


def _tc_final_body(p, base, dis, W, b, mu_ref, ls_ref):
    disv = dis[...]
    pv = p[...]
    z = (pv[0] + pv[1] + base[...]) * disv
    o = jnp.dot(z, W[...], preferred_element_type=jnp.float32,
                precision=lax.Precision.HIGHEST) + b[...][None, :]
    mu_ref[...] = o[:, :OUT]
    ls_ref[...] = o[:, OUT:]


_tc_final = pl.pallas_call(
    _tc_final_body,
    grid=(G,),
    in_specs=_Y_SPECS + [
        pl.BlockSpec((R, C), lambda i: (i, 0)),
        pl.BlockSpec((R, 1), lambda i: (i, 0)),
        pl.BlockSpec((C, C), lambda i: (0, 0)),
        pl.BlockSpec((C,), lambda i: (0,)),
    ],
    out_specs=[
        pl.BlockSpec((R, OUT), lambda i: (i, 0)),
        pl.BlockSpec((R, OUT), lambda i: (i, 0)),
    ],
    out_shape=[
        jax.ShapeDtypeStruct((N, OUT), jnp.float32),
        jax.ShapeDtypeStruct((N, OUT), jnp.float32),
    ],
)


def kernel(x, edge_index, W1, b1, W_mu, b_mu, W_ls, b_ls):
    ei = edge_index.astype(jnp.int32)
    src, dst = ei[0], ei[1]
    pad = E_PAD - src.shape[0]
    src_f = jnp.concatenate([src, jnp.zeros((pad,), jnp.int32)])
    dst_f = jnp.concatenate([dst, jnp.full((pad,), N, jnp.int32)])
    dst_p = dst_f.reshape(NW, CH, CHUNK)
    nchunks = E_PAD // CHUNK
    ei_p = jnp.stack([src_f.reshape(nchunks, CHUNK),
                      dst_f.reshape(nchunks, CHUNK)],
                     axis=1).reshape(NW, CH, 2, CHUNK)

    degp = _sc_deg(dst_p)
    d0 = degp[0, :N, 0:1]
    d1 = degp[1, :N, 0:1]
    xs, dis = _tc_prescale(d0, d1, x)

    y1 = _sc_agg(xs, ei_p)
    hs = _tc_mid(y1, xs, dis, W1, b1)

    y2 = _sc_agg(hs, ei_p)
    Wc = jnp.concatenate([W_mu, W_ls], axis=1)
    bc = jnp.concatenate([b_mu, b_ls])
    mu, ls = _tc_final(y2, hs, dis, Wc, bc)
    return mu, ls
